# baseline (speedup 1.0000x reference)
import jax
import jax.numpy as jnp
from jax import lax
from jax.experimental import pallas as pl
from jax.experimental.pallas import tpu as pltpu
from jax.experimental.pallas import tpu_sc as plsc

N_VOCAB = 100000
N_POSITION = 4096
D_MODEL = 768
SEQ = 4096
BATCH = 4

NC = 2
NS = 16
NW = NC * NS

N_ROWS = SEQ * BATCH
RPW = N_ROWS // NW
CH = 32
NCHUNK = RPW // CH
LANES = 16
NVEC = D_MODEL // LANES


def _sc_body(seq_hbm, posidx_hbm, emb_hbm, pos_hbm, out_hbm,
             idx_t, idx_p, tok_v, pos_v, sem_t, sem_p, sem_o):
    cid = lax.axis_index("c")
    sid = lax.axis_index("s")
    wid = sid * NC + cid

    pltpu.sync_copy(seq_hbm.at[wid], idx_t)
    pltpu.sync_copy(posidx_hbm.at[wid], idx_p)

    base = wid * RPW

    tok_d = [None] * NCHUNK
    pos_d = [None] * NCHUNK
    out_d = [None] * NCHUNK

    def gathers(c, b):
        tok_d[c] = pltpu.async_copy(emb_hbm.at[idx_t.at[c]], tok_v.at[b],
                                    sem_t)
        pos_d[c] = pltpu.async_copy(pos_hbm.at[idx_p.at[c]], pos_v.at[b],
                                    sem_p)

    gathers(0, 0)

    for c in range(NCHUNK):
        b = c & 1
        if c + 1 < NCHUNK:
            if c >= 1:
                out_d[c - 1].wait()
            gathers(c + 1, 1 - b)
        tok_d[c].wait()
        pos_d[c].wait()

        def row_body(r, carry):
            for j in range(NVEC):
                sl = pl.ds(j * LANES, LANES)
                plsc.addupdate(tok_v.at[b, r, sl], pos_v[b, r, sl])
            return carry

        lax.fori_loop(0, CH, row_body, 0, unroll=4)

        off = pl.multiple_of(base + c * CH, CH)
        out_d[c] = pltpu.async_copy(tok_v.at[b], out_hbm.at[pl.ds(off, CH)],
                                    sem_o)

    out_d[NCHUNK - 2].wait()
    out_d[NCHUNK - 1].wait()


@jax.jit
def kernel(input_seq, input_positions, emb_table, pos_table):
    seq_flat = input_seq.reshape(NW, NCHUNK, CH)
    pos_flat = input_positions.reshape(NW, NCHUNK, CH)

    mesh = plsc.VectorSubcoreMesh(core_axis_name="c", subcore_axis_name="s",
                                  num_cores=NC, num_subcores=NS)
    out = pl.kernel(
        _sc_body,
        out_type=jax.ShapeDtypeStruct((N_ROWS, D_MODEL), jnp.float32),
        mesh=mesh,
        scratch_types=[
            pltpu.VMEM((NCHUNK, CH), jnp.int32),
            pltpu.VMEM((NCHUNK, CH), jnp.int32),
            pltpu.VMEM((2, CH, D_MODEL), jnp.float32),
            pltpu.VMEM((2, CH, D_MODEL), jnp.float32),
            pltpu.SemaphoreType.DMA,
            pltpu.SemaphoreType.DMA,
            pltpu.SemaphoreType.DMA,
        ],
    )(seq_flat, pos_flat, emb_table, pos_table)
    return out.reshape(SEQ, BATCH, D_MODEL)


# speedup vs baseline: 1.0217x; 1.0217x over previous
import jax
import jax.numpy as jnp
from jax import lax
from jax.experimental import pallas as pl
from jax.experimental.pallas import tpu as pltpu
from jax.experimental.pallas import tpu_sc as plsc

N_VOCAB = 100000
N_POSITION = 4096
D_MODEL = 768
SEQ = 4096
BATCH = 4

NC = 2
NS = 16
NW = NC * NS

N_ROWS = SEQ * BATCH
RPW = N_ROWS // NW
CH = 32
NCHUNK = RPW // CH
LANES = 16
NVEC = D_MODEL // LANES


def _sc_body(seq_hbm, posidx_hbm, emb_hbm, pos_hbm, out_hbm,
             idx_t, idx_p, tok_v, pos_v, sem_t, sem_p, sem_o):
    cid = lax.axis_index("c")
    sid = lax.axis_index("s")
    wid = sid * NC + cid

    pltpu.sync_copy(seq_hbm.at[wid], idx_t)
    pltpu.sync_copy(posidx_hbm.at[wid], idx_p)

    base = wid * RPW

    tok_d = [None] * NCHUNK
    pos_d = [None] * NCHUNK
    out_d = [None] * NCHUNK

    def gathers(c, b):
        tok_d[c] = pltpu.async_copy(emb_hbm.at[idx_t.at[c]], tok_v.at[b],
                                    sem_t)
        pos_d[c] = pltpu.async_copy(pos_hbm.at[idx_p.at[c]], pos_v.at[b],
                                    sem_p)

    gathers(0, 0)

    for c in range(NCHUNK):
        b = c & 1
        if c + 1 < NCHUNK:
            if c >= 1:
                out_d[c - 1].wait()
            gathers(c + 1, 1 - b)
        tok_d[c].wait()
        pos_d[c].wait()

        def row_body(r, carry):
            for j in range(NVEC):
                sl = pl.ds(j * LANES, LANES)
                plsc.addupdate(tok_v.at[b, r, sl], pos_v[b, r, sl])
            return carry

        lax.fori_loop(0, CH, row_body, 0, unroll=2)

        off = pl.multiple_of(base + c * CH, CH)
        out_d[c] = pltpu.async_copy(tok_v.at[b], out_hbm.at[pl.ds(off, CH)],
                                    sem_o)

    out_d[NCHUNK - 2].wait()
    out_d[NCHUNK - 1].wait()


@jax.jit
def kernel(input_seq, input_positions, emb_table, pos_table):
    seq_flat = input_seq.reshape(NW, NCHUNK, CH)
    pos_flat = input_positions.reshape(NW, NCHUNK, CH)

    mesh = plsc.VectorSubcoreMesh(core_axis_name="c", subcore_axis_name="s",
                                  num_cores=NC, num_subcores=NS)
    out = pl.kernel(
        _sc_body,
        out_type=jax.ShapeDtypeStruct((N_ROWS, D_MODEL), jnp.float32),
        mesh=mesh,
        scratch_types=[
            pltpu.VMEM((NCHUNK, CH), jnp.int32),
            pltpu.VMEM((NCHUNK, CH), jnp.int32),
            pltpu.VMEM((2, CH, D_MODEL), jnp.float32),
            pltpu.VMEM((2, CH, D_MODEL), jnp.float32),
            pltpu.SemaphoreType.DMA,
            pltpu.SemaphoreType.DMA,
            pltpu.SemaphoreType.DMA,
        ],
    )(seq_flat, pos_flat, emb_table, pos_table)
    return out.reshape(SEQ, BATCH, D_MODEL)
